# Initial kernel scaffold; baseline (speedup 1.0000x reference)
import jax
import jax.numpy as jnp
from jax.experimental import pallas as pl


def kernel(inputs, W, b):
    raise NotImplementedError("write your pallas kernel here")


# speedup vs baseline: 2.3684x; 2.3684x over previous
import functools

import jax
import jax.numpy as jnp
from jax import lax
from jax.experimental import pallas as pl
from jax.experimental.pallas import tpu as pltpu

_S = 8
_L = 512


def _split_body(x_ref, w_ref, b_ref, o_ref):
    T = x_ref.shape[1]
    x = x_ref[0]
    pooled = jnp.mean(x, axis=0, keepdims=True)
    offs = (
        jnp.dot(pooled, w_ref[...], preferred_element_type=jnp.float32)
        + b_ref[...]
    )
    col = lax.broadcasted_iota(jnp.int32, (1, 2 * _S), 1)
    offs_c = jnp.clip(offs, 0.0, float(_L - 1))
    row = lax.broadcasted_iota(jnp.int32, (_L, 1), 0)
    for s in range(_S):
        start_off = jnp.sum(jnp.where(col == s, offs_c, 0.0)).astype(jnp.int32)
        end_off = jnp.sum(jnp.where(col == _S + s, offs_c, 0.0)).astype(jnp.int32)
        start_i = jnp.clip(s * _L + start_off, 0, T - _L)
        end_i = jnp.clip(s * _L + _L + end_off, start_i, T)
        length = end_i - start_i
        base = jnp.minimum((start_i // 8) * 8, T - (_L + 8))
        shift = start_i - base
        w0 = x_ref[0, pl.ds(pl.multiple_of(base, 8), _L + 8), :]
        for k in range(9):
            @pl.when(shift == k)
            def _(s=s, k=k, w0=w0, length=length):
                o_ref[0, s] = jnp.where(row < length, w0[k:k + _L, :], 0.0)


def kernel(inputs, W, b):
    B, T, D = inputs.shape
    b2 = b.reshape(1, 2 * _S)
    return pl.pallas_call(
        _split_body,
        grid=(B,),
        in_specs=[
            pl.BlockSpec((1, T, D), lambda i: (i, 0, 0)),
            pl.BlockSpec((D, 2 * _S), lambda i: (0, 0)),
            pl.BlockSpec((1, 2 * _S), lambda i: (0, 0)),
        ],
        out_specs=pl.BlockSpec((1, _S, _L, D), lambda i: (i, 0, 0, 0)),
        out_shape=jax.ShapeDtypeStruct((B, _S, _L, D), jnp.float32),
    )(inputs, W, b2)
